# baseline (speedup 1.0000x reference)
import functools

import jax
import jax.numpy as jnp
from jax import lax
from jax.experimental import pallas as pl
from jax.experimental.pallas import tpu as pltpu
from jax.experimental.pallas import tpu_sc as plsc

_NC = 2
_NS = 16
_L = 16


@functools.lru_cache(maxsize=None)
def _make_sc_kernel(B, T, D):
    R = T + 1
    RP = R + 7
    NW = _NC * _NS
    bw = B // NW

    mesh = plsc.VectorSubcoreMesh(core_axis_name="c", subcore_axis_name="s")

    @functools.partial(
        pl.kernel,
        mesh=mesh,
        out_type=jax.ShapeDtypeStruct((B, R, D), jnp.float32),
        scratch_types=[
            pltpu.VMEM((RP,), jnp.int32),
            pltpu.VMEM((RP,), jnp.float32),
            pltpu.VMEM((RP, D), jnp.float32),
            pltpu.VMEM((RP, D), jnp.float32),
            pltpu.SemaphoreType.DMA,
        ],
        compiler_params=pltpu.CompilerParams(use_tc_tiling_on_sc=False),
    )
    def k(emb_hbm, idx_hbm, rat_hbm, pos_hbm, out_hbm,
          idx_v, rat_v, pos_v, rows_v, sem):
        wid = lax.axis_index("s") * _NC + lax.axis_index("c")
        pltpu.sync_copy(pos_hbm, pos_v)

        def batch_body(i, carry):
            b = wid * bw + i
            pltpu.sync_copy(idx_hbm.at[b], idx_v)
            pltpu.sync_copy(rat_hbm.at[b], rat_v)
            cp0 = pltpu.async_copy(
                emb_hbm.at[idx_v.at[pl.ds(0, 104)]],
                rows_v.at[pl.ds(0, 104)], sem)
            cp1 = pltpu.async_copy(
                emb_hbm.at[idx_v.at[pl.ds(104, R - 104)]],
                rows_v.at[pl.ds(104, R - 104)], sem)
            cp0.wait()
            cp1.wait()

            def chunk_body(tc, carry2):
                base = tc * _L
                rv = rat_v[pl.ds(base, _L)]
                for j in range(_L):
                    r = rv[j]
                    t = base + j
                    for c in range(D // _L):
                        sl = pl.ds(c * _L, _L)
                        rows_v[t, sl] = (rows_v[t, sl] + pos_v[t, sl]) * r
                return carry2

            lax.fori_loop(0, RP // _L, chunk_body, 0)
            pltpu.sync_copy(rows_v.at[pl.ds(0, R)], out_hbm.at[b])
            return carry

        lax.fori_loop(0, bw, batch_body, 0)

    return k


def kernel(user_id, product_history, target_product_id,
           product_history_ratings, emb_weights, pos_weights):
    B, T = product_history_ratings.shape
    D = emb_weights.shape[1]
    pad = 7
    idx_full = jnp.concatenate([
        product_history.astype(jnp.int32),
        target_product_id.astype(jnp.int32).reshape(B, 1),
        jnp.zeros((B, pad), jnp.int32),
    ], axis=1)
    rat_full = jnp.concatenate([
        product_history_ratings,
        jnp.ones((B, 1), jnp.float32),
        jnp.zeros((B, pad), jnp.float32),
    ], axis=1)
    pos_pad = jnp.concatenate([
        pos_weights,
        jnp.zeros((1 + pad, D), jnp.float32),
    ], axis=0)
    return _make_sc_kernel(B, T, D)(emb_weights, idx_full, rat_full, pos_pad)
